# baseline (speedup 1.0000x reference)
import jax
import jax.numpy as jnp
from jax import lax
from jax.experimental import pallas as pl
from jax.experimental.pallas import tpu as pltpu
from jax.experimental.pallas import tpu_sc as plsc

N_ROWS = 100000
N_COLS = 128
N_SEG = 256
NC = 2
NS = 16
COLS_PER_CORE = N_COLS // NC
CHUNK = 400
N_REG = 8
N_CHUNKS = N_ROWS // CHUNK
SUB = 80
SUBS_PER_CHUNK = CHUNK // SUB
SEG_PER_TILE = N_SEG // NS
MAX_CHUNKS_PER_TILE = (N_CHUNKS + NS - 1) // NS
N_PAIRS = (MAX_CHUNKS_PER_TILE + 1) // 2
S_STREAM = 5


def _body(feat_hbm, ids_hbm, out_hbm,
          feat0_v, feat1_v, ids_all_v, hist_v, idx2_v,
          acc_v, cnt_v, outb_v,
          semf0, semf1, sems0, sems1, semi, semz,
          accum_sh, counts_sh):
    c = lax.axis_index("c")
    t = lax.axis_index("s")
    col0 = c * COLS_PER_CORE
    feat_bufs = (feat0_v, feat1_v)
    load_sems = (semf0, semf1)
    scat_sems = (sems0, sems1)

    n_my_chunks = (N_CHUNKS - t + NS - 1) // NS

    def feat_copy(i, b):
        g = t + i * NS
        return pltpu.make_async_copy(
            feat_hbm.at[pl.ds(g * CHUNK, CHUNK), pl.ds(col0, COLS_PER_CORE)],
            feat_bufs[b], load_sems[b])

    def scat_start(i, b, j):
        pltpu.async_copy(
            feat_bufs[b].at[pl.ds(j * SUB, SUB)],
            accum_sh.at[ids_all_v.at[i * SUBS_PER_CHUNK + j]],
            scat_sems[b], add=True)

    def scat_wait(i, b, j):
        pltpu.make_async_copy(
            feat_bufs[b].at[pl.ds(j * SUB, SUB)],
            accum_sh.at[ids_all_v.at[i * SUBS_PER_CHUNK + j]],
            scat_sems[b]).wait()

    feat_copy(0, 0).start()

    ones16 = jnp.full((16,), 1.0, jnp.float32)
    zeros16 = jnp.zeros((16,), jnp.float32)
    lanes = lax.iota(jnp.int32, 16)
    zlanes = jnp.zeros((16,), jnp.int32)
    colvs = [lanes + 16 * k for k in range(COLS_PER_CORE // 16)]
    def init_zero(r, carry):
        for j in range(COLS_PER_CORE // 16):
            acc_v[r, pl.ds(j * 16, 16)] = zeros16
        return carry
    lax.fori_loop(0, N_SEG, init_zero, 0)
    for s in range(N_SEG // 16):
        for j in range(16):
            hist_v[s * 16 + j, pl.ds(0, 16)] = zeros16
    for r in range(2):
        for k in range(8):
            idx2_v[r, pl.ds(k * 16, 16)] = lanes + (r * 128 + k * 16)

    def ids_load(i, carry):
        g = jnp.minimum(t + i * NS, N_CHUNKS - 1)
        pltpu.async_copy(
            ids_hbm.at[pl.ds(g * SUBS_PER_CHUNK, SUBS_PER_CHUNK)],
            ids_all_v.at[pl.ds(i * SUBS_PER_CHUNK, SUBS_PER_CHUNK)],
            semi)
        return carry
    lax.fori_loop(0, MAX_CHUNKS_PER_TILE, ids_load, 0)
    pltpu.make_async_copy(
        ids_hbm.at[pl.ds(0, MAX_CHUNKS_PER_TILE * SUBS_PER_CHUNK)],
        ids_all_v, semi).wait()

    seg0 = t * SEG_PER_TILE
    my_reg = (t // 2) * N_SEG
    zoff = my_reg + lax.rem(t, 2) * (N_SEG // 2)
    for q in range(8):
        pltpu.async_copy(acc_v.at[pl.ds(0, SEG_PER_TILE)],
                         accum_sh.at[pl.ds(zoff + q * SEG_PER_TILE,
                                           SEG_PER_TILE)], semz)
    pltpu.sync_copy(hist_v.at[pl.ds(0, SEG_PER_TILE)],
                    counts_sh.at[pl.ds(seg0, SEG_PER_TILE)])
    for q in range(8):
        pltpu.make_async_copy(acc_v.at[pl.ds(0, SEG_PER_TILE)],
                              accum_sh.at[pl.ds(zoff + q * SEG_PER_TILE,
                                                SEG_PER_TILE)], semz).wait()
    plsc.subcore_barrier()

    def hist_chunk(i):
        for j in range(SUBS_PER_CHUNK):
            r = i * SUBS_PER_CHUNK + j
            for k in range(SUB // 16):
                idv = ids_all_v[r, pl.ds(k * 16, 16)]
                plsc.addupdate_scatter(hist_v, [idv, zlanes], ones16)
                if j < S_STREAM:
                    ids_all_v[r, pl.ds(k * 16, 16)] = idv + my_reg

    def valu_subs(i, b):
        def sub_body(j, carry):
            def group(m, carry2):
                r = i * SUBS_PER_CHUNK + j
                idvec = ids_all_v[r, pl.ds(m * 16, 16)]
                row0 = j * SUB + m * 16
                for l in range(16):
                    bid = jnp.full((16,), idvec[l], jnp.int32)
                    for k in range(COLS_PER_CORE // 16):
                        fv = feat_bufs[b][row0 + l, pl.ds(k * 16, 16)]
                        plsc.addupdate_scatter(acc_v, [bid, colvs[k]], fv)
                return carry2
            lax.fori_loop(0, SUB // 16, group, 0)
            return carry
        lax.fori_loop(S_STREAM, SUBS_PER_CHUNK, sub_body, 0)

    def pair_body(p, carry):
        for b in range(2):
            i = 2 * p + b

            @pl.when(i < n_my_chunks)
            def _process():
                hist_chunk(i)
                feat_copy(i, b).wait()
                for j in range(S_STREAM):
                    scat_start(i, b, j)

                @pl.when(i > 0)
                def _drain_other():
                    for j in range(S_STREAM):
                        scat_wait(i - 1, 1 - b, j)

                @pl.when(i + 1 < n_my_chunks)
                def _prefetch():
                    feat_copy(i + 1, 1 - b).start()

                valu_subs(i, b)
        return carry
    lax.fori_loop(0, N_PAIRS, pair_body, 0)

    for nt_par in range(2):
        @pl.when(lax.rem(n_my_chunks, 2) == nt_par)
        def _drain_last():
            b_last = 1 - nt_par
            for j in range(S_STREAM):
                scat_wait(n_my_chunks - 1, b_last, j)

    for r in range(2):
        pltpu.sync_copy(hist_v.at[pl.ds(r * 128, 128)],
                        counts_sh.at[idx2_v.at[r]], add=True)
    for r in range(2):
        for k in range(8):
            idx2_v[r, pl.ds(k * 16, 16)] = \
                idx2_v[r, pl.ds(k * 16, 16)] + my_reg
    for r in range(2):
        pltpu.sync_copy(acc_v.at[pl.ds(r * 128, 128)],
                        accum_sh.at[idx2_v.at[r]], add=True)

    plsc.subcore_barrier()

    for r in range(N_REG):
        pltpu.async_copy(
            accum_sh.at[pl.ds(r * N_SEG + seg0, SEG_PER_TILE)],
            feat0_v.at[pl.ds(r * SEG_PER_TILE, SEG_PER_TILE)], semi)
    cnt_cp = pltpu.make_async_copy(counts_sh.at[pl.ds(seg0, SEG_PER_TILE)],
                                   cnt_v, semi)
    cnt_cp.start()
    cnt_cp.wait()
    pltpu.make_async_copy(
        accum_sh.at[pl.ds(0, N_REG * SEG_PER_TILE)],
        feat0_v.at[pl.ds(0, N_REG * SEG_PER_TILE)], semi).wait()
    for s in range(SEG_PER_TILE):
        for j in range(COLS_PER_CORE // 16):
            outb_v[s, pl.ds(j * 16, 16)] = feat0_v[s, pl.ds(j * 16, 16)]

    def sum_regions(r, carry):
        for s in range(SEG_PER_TILE):
            for j in range(COLS_PER_CORE // 16):
                plsc.addupdate(
                    outb_v.at[s, pl.ds(j * 16, 16)],
                    feat0_v[r * SEG_PER_TILE + s, pl.ds(j * 16, 16)])
        return carry
    lax.fori_loop(1, N_REG, sum_regions, 0)

    for s in range(SEG_PER_TILE):
        cnt_row = cnt_v[s, pl.ds(0, 16)]
        cntv = jnp.full((16,), cnt_row[0], jnp.float32)
        inv = 1.0 / jnp.maximum(cntv, 1.0)
        for j in range(COLS_PER_CORE // 16):
            outb_v[s, pl.ds(j * 16, 16)] = outb_v[s, pl.ds(j * 16, 16)] * inv
    pltpu.sync_copy(outb_v,
                    out_hbm.at[pl.ds(seg0, SEG_PER_TILE),
                               pl.ds(col0, COLS_PER_CORE)])


def kernel(features, graph_ids):
    ids = graph_ids.astype(jnp.int32).reshape(N_ROWS // SUB, SUB)
    mesh = plsc.VectorSubcoreMesh(core_axis_name="c", subcore_axis_name="s")
    f = pl.kernel(
        _body,
        out_type=jax.ShapeDtypeStruct((N_SEG, N_COLS), jnp.float32),
        mesh=mesh,
        scratch_types=[
            pltpu.VMEM((CHUNK, COLS_PER_CORE), jnp.float32),
            pltpu.VMEM((CHUNK, COLS_PER_CORE), jnp.float32),
            pltpu.VMEM((MAX_CHUNKS_PER_TILE * SUBS_PER_CHUNK, SUB),
                       jnp.int32),
            pltpu.VMEM((N_SEG, 16), jnp.float32),
            pltpu.VMEM((2, 128), jnp.int32),
            pltpu.VMEM((N_SEG, COLS_PER_CORE), jnp.float32),
            pltpu.VMEM((SEG_PER_TILE, 16), jnp.float32),
            pltpu.VMEM((SEG_PER_TILE, COLS_PER_CORE), jnp.float32),
            pltpu.SemaphoreType.DMA,
            pltpu.SemaphoreType.DMA,
            pltpu.SemaphoreType.DMA,
            pltpu.SemaphoreType.DMA,
            pltpu.SemaphoreType.DMA,
            pltpu.SemaphoreType.DMA,
            pltpu.VMEM_SHARED((N_REG * N_SEG, COLS_PER_CORE), jnp.float32),
            pltpu.VMEM_SHARED((N_SEG, 16), jnp.float32),
        ],
        compiler_params=pltpu.CompilerParams(use_tc_tiling_on_sc=False,
                                             needs_layout_passes=False,
                                             skip_device_barrier=True),
    )
    return f(features, ids)


# speedup vs baseline: 1.0468x; 1.0468x over previous
import jax
import jax.numpy as jnp
from jax import lax
from jax.experimental import pallas as pl
from jax.experimental.pallas import tpu as pltpu
from jax.experimental.pallas import tpu_sc as plsc

N_ROWS = 100000
N_COLS = 128
N_SEG = 256
NC = 2
NS = 16
COLS_PER_CORE = N_COLS // NC
CHUNK = 400
N_REG = 8
N_CHUNKS = N_ROWS // CHUNK
SUB = 80
SUBS_PER_CHUNK = CHUNK // SUB
SEG_PER_TILE = N_SEG // NS
MAX_CHUNKS_PER_TILE = (N_CHUNKS + NS - 1) // NS
N_PAIRS = (MAX_CHUNKS_PER_TILE + 1) // 2
S_STREAM = 4


def _body(feat_hbm, ids_hbm, out_hbm,
          feat0_v, feat1_v, feat2_v, ids_all_v, hist_v, idx2_v,
          acc_v, cnt_v, outb_v,
          semf0, semf1, semf2, sems0, sems1, sems2, semi, semz,
          accum_sh, counts_sh):
    c = lax.axis_index("c")
    t = lax.axis_index("s")
    col0 = c * COLS_PER_CORE
    feat_bufs = (feat0_v, feat1_v, feat2_v)
    load_sems = (semf0, semf1, semf2)
    scat_sems = (sems0, sems1, sems2)

    n_my_chunks = (N_CHUNKS - t + NS - 1) // NS

    def feat_copy(i, b):
        g = t + i * NS
        return pltpu.make_async_copy(
            feat_hbm.at[pl.ds(g * CHUNK, CHUNK), pl.ds(col0, COLS_PER_CORE)],
            feat_bufs[b], load_sems[b])

    def scat_start(i, b, j):
        pltpu.async_copy(
            feat_bufs[b].at[pl.ds(j * SUB, SUB)],
            accum_sh.at[ids_all_v.at[i * SUBS_PER_CHUNK + j]],
            scat_sems[b], add=True)

    def scat_wait(i, b, j):
        pltpu.make_async_copy(
            feat_bufs[b].at[pl.ds(j * SUB, SUB)],
            accum_sh.at[ids_all_v.at[i * SUBS_PER_CHUNK + j]],
            scat_sems[b]).wait()

    feat_copy(0, 0).start()
    feat_copy(1, 1).start()

    ones16 = jnp.full((16,), 1.0, jnp.float32)
    zeros16 = jnp.zeros((16,), jnp.float32)
    lanes = lax.iota(jnp.int32, 16)
    zlanes = jnp.zeros((16,), jnp.int32)
    colvs = [lanes + 16 * k for k in range(COLS_PER_CORE // 16)]
    def init_zero(r, carry):
        for j in range(COLS_PER_CORE // 16):
            acc_v[r, pl.ds(j * 16, 16)] = zeros16
        return carry
    lax.fori_loop(0, N_SEG, init_zero, 0)
    for s in range(N_SEG // 16):
        for j in range(16):
            hist_v[s * 16 + j, pl.ds(0, 16)] = zeros16
    for r in range(2):
        for k in range(8):
            idx2_v[r, pl.ds(k * 16, 16)] = lanes + (r * 128 + k * 16)

    def ids_load(i, carry):
        g = jnp.minimum(t + i * NS, N_CHUNKS - 1)
        pltpu.async_copy(
            ids_hbm.at[pl.ds(g * SUBS_PER_CHUNK, SUBS_PER_CHUNK)],
            ids_all_v.at[pl.ds(i * SUBS_PER_CHUNK, SUBS_PER_CHUNK)],
            semi)
        return carry
    lax.fori_loop(0, MAX_CHUNKS_PER_TILE, ids_load, 0)
    pltpu.make_async_copy(
        ids_hbm.at[pl.ds(0, MAX_CHUNKS_PER_TILE * SUBS_PER_CHUNK)],
        ids_all_v, semi).wait()

    seg0 = t * SEG_PER_TILE
    my_reg = (t // 2) * N_SEG
    zoff = my_reg + lax.rem(t, 2) * (N_SEG // 2)
    for q in range(8):
        pltpu.async_copy(acc_v.at[pl.ds(0, SEG_PER_TILE)],
                         accum_sh.at[pl.ds(zoff + q * SEG_PER_TILE,
                                           SEG_PER_TILE)], semz)
    pltpu.sync_copy(hist_v.at[pl.ds(0, SEG_PER_TILE)],
                    counts_sh.at[pl.ds(seg0, SEG_PER_TILE)])
    for q in range(8):
        pltpu.make_async_copy(acc_v.at[pl.ds(0, SEG_PER_TILE)],
                              accum_sh.at[pl.ds(zoff + q * SEG_PER_TILE,
                                                SEG_PER_TILE)], semz).wait()
    plsc.subcore_barrier()

    def hist_chunk(i):
        for j in range(SUBS_PER_CHUNK):
            r = i * SUBS_PER_CHUNK + j
            for k in range(SUB // 16):
                idv = ids_all_v[r, pl.ds(k * 16, 16)]
                plsc.addupdate_scatter(hist_v, [idv, zlanes], ones16)
                if j < S_STREAM:
                    ids_all_v[r, pl.ds(k * 16, 16)] = idv + my_reg

    def valu_subs(i, b):
        def sub_body(j, carry):
            def group(m, carry2):
                r = i * SUBS_PER_CHUNK + j
                idvec = ids_all_v[r, pl.ds(m * 16, 16)]
                row0 = j * SUB + m * 16
                for l in range(16):
                    bid = jnp.full((16,), idvec[l], jnp.int32)
                    for k in range(COLS_PER_CORE // 16):
                        fv = feat_bufs[b][row0 + l, pl.ds(k * 16, 16)]
                        plsc.addupdate_scatter(acc_v, [bid, colvs[k]], fv)
                return carry2
            lax.fori_loop(0, SUB // 16, group, 0)
            return carry
        lax.fori_loop(S_STREAM, SUBS_PER_CHUNK, sub_body, 0)

    def tri_body(p, carry):
        for b in range(3):
            i = 3 * p + b

            @pl.when(i < n_my_chunks)
            def _process():
                hist_chunk(i)
                feat_copy(i, b).wait()
                for j in range(S_STREAM):
                    scat_start(i, b, j)

                @pl.when(i > 0)
                def _drain_other():
                    for j in range(S_STREAM):
                        scat_wait(i - 1, (b + 2) % 3, j)

                @pl.when(i + 2 < n_my_chunks)
                def _prefetch():
                    feat_copy(i + 2, (b + 2) % 3).start()

                valu_subs(i, b)
        return carry
    lax.fori_loop(0, (MAX_CHUNKS_PER_TILE + 2) // 3, tri_body, 0)

    for nt_par in range(3):
        @pl.when(lax.rem(n_my_chunks, 3) == nt_par)
        def _drain_last():
            b_last = (nt_par + 2) % 3
            for j in range(S_STREAM):
                scat_wait(n_my_chunks - 1, b_last, j)

    for r in range(2):
        pltpu.sync_copy(hist_v.at[pl.ds(r * 128, 128)],
                        counts_sh.at[idx2_v.at[r]], add=True)
    for r in range(2):
        for k in range(8):
            idx2_v[r, pl.ds(k * 16, 16)] = \
                idx2_v[r, pl.ds(k * 16, 16)] + my_reg
    for r in range(2):
        pltpu.sync_copy(acc_v.at[pl.ds(r * 128, 128)],
                        accum_sh.at[idx2_v.at[r]], add=True)

    plsc.subcore_barrier()

    for r in range(N_REG):
        pltpu.async_copy(
            accum_sh.at[pl.ds(r * N_SEG + seg0, SEG_PER_TILE)],
            feat0_v.at[pl.ds(r * SEG_PER_TILE, SEG_PER_TILE)], semi)
    cnt_cp = pltpu.make_async_copy(counts_sh.at[pl.ds(seg0, SEG_PER_TILE)],
                                   cnt_v, semi)
    cnt_cp.start()
    cnt_cp.wait()
    pltpu.make_async_copy(
        accum_sh.at[pl.ds(0, N_REG * SEG_PER_TILE)],
        feat0_v.at[pl.ds(0, N_REG * SEG_PER_TILE)], semi).wait()
    for s in range(SEG_PER_TILE):
        for j in range(COLS_PER_CORE // 16):
            outb_v[s, pl.ds(j * 16, 16)] = feat0_v[s, pl.ds(j * 16, 16)]

    def sum_regions(r, carry):
        for s in range(SEG_PER_TILE):
            for j in range(COLS_PER_CORE // 16):
                plsc.addupdate(
                    outb_v.at[s, pl.ds(j * 16, 16)],
                    feat0_v[r * SEG_PER_TILE + s, pl.ds(j * 16, 16)])
        return carry
    lax.fori_loop(1, N_REG, sum_regions, 0)

    for s in range(SEG_PER_TILE):
        cnt_row = cnt_v[s, pl.ds(0, 16)]
        cntv = jnp.full((16,), cnt_row[0], jnp.float32)
        inv = 1.0 / jnp.maximum(cntv, 1.0)
        for j in range(COLS_PER_CORE // 16):
            outb_v[s, pl.ds(j * 16, 16)] = outb_v[s, pl.ds(j * 16, 16)] * inv
    pltpu.sync_copy(outb_v,
                    out_hbm.at[pl.ds(seg0, SEG_PER_TILE),
                               pl.ds(col0, COLS_PER_CORE)])


def kernel(features, graph_ids):
    ids = graph_ids.astype(jnp.int32).reshape(N_ROWS // SUB, SUB)
    mesh = plsc.VectorSubcoreMesh(core_axis_name="c", subcore_axis_name="s")
    f = pl.kernel(
        _body,
        out_type=jax.ShapeDtypeStruct((N_SEG, N_COLS), jnp.float32),
        mesh=mesh,
        scratch_types=[
            pltpu.VMEM((CHUNK, COLS_PER_CORE), jnp.float32),
            pltpu.VMEM((CHUNK, COLS_PER_CORE), jnp.float32),
            pltpu.VMEM((CHUNK, COLS_PER_CORE), jnp.float32),
            pltpu.VMEM((MAX_CHUNKS_PER_TILE * SUBS_PER_CHUNK, SUB),
                       jnp.int32),
            pltpu.VMEM((N_SEG, 16), jnp.float32),
            pltpu.VMEM((2, 128), jnp.int32),
            pltpu.VMEM((N_SEG, COLS_PER_CORE), jnp.float32),
            pltpu.VMEM((SEG_PER_TILE, 16), jnp.float32),
            pltpu.VMEM((SEG_PER_TILE, COLS_PER_CORE), jnp.float32),
            pltpu.SemaphoreType.DMA,
            pltpu.SemaphoreType.DMA,
            pltpu.SemaphoreType.DMA,
            pltpu.SemaphoreType.DMA,
            pltpu.SemaphoreType.DMA,
            pltpu.SemaphoreType.DMA,
            pltpu.SemaphoreType.DMA,
            pltpu.SemaphoreType.DMA,
            pltpu.VMEM_SHARED((N_REG * N_SEG, COLS_PER_CORE), jnp.float32),
            pltpu.VMEM_SHARED((N_SEG, 16), jnp.float32),
        ],
        compiler_params=pltpu.CompilerParams(use_tc_tiling_on_sc=False,
                                             needs_layout_passes=False,
                                             skip_device_barrier=True),
    )
    return f(features, ids)
